# Initial kernel scaffold; baseline (speedup 1.0000x reference)
import jax
import jax.numpy as jnp
from jax.experimental import pallas as pl


def kernel(x, Uw, Ub, Vw, Vb, gamma, beta):
    raise NotImplementedError("write your pallas kernel here")


# speedup vs baseline: 1.1591x; 1.1591x over previous
import jax
import jax.numpy as jnp
from jax import lax
from jax.experimental import pallas as pl
from jax.experimental.pallas import tpu as pltpu

_K = 4
_EPS = 1e-5
_HP = lax.Precision.HIGHEST


def _graph_kernel(x_ref, wuv_ref, buv_ref, y_ref, stats_ref):
    x = x_ref[...]
    N, C = x.shape

    uv = jnp.dot(x, wuv_ref[...],
                 preferred_element_type=jnp.float32) + buv_ref[...]
    Ux = uv[:, :C]
    Vx = uv[:, C:]

    si = lax.dot_general(x, x, (((1,), (1,)), ((), ())),
                         precision=_HP,
                         preferred_element_type=jnp.float32)

    work = si
    cnt = jnp.zeros((N, 1), jnp.float32)
    thr = jnp.full((N, 1), -jnp.inf, jnp.float32)
    kf = jnp.float32(_K)
    for r in range(_K):
        m = jnp.max(work, axis=-1, keepdims=True)
        eq = work == m
        c = jnp.sum(eq.astype(jnp.float32), axis=-1, keepdims=True)
        ncnt = cnt + c
        thr = jnp.where((cnt < kf) & (ncnt >= kf), m, thr)
        cnt = ncnt
        if r + 1 < _K:
            work = jnp.where(eq, -jnp.inf, work)
    adj = (si >= thr).astype(jnp.float32)

    deg = jnp.sum(adj, axis=-1, keepdims=True)
    dinv = lax.rsqrt(deg)
    agg = dinv * jnp.dot(adj, dinv * Vx, preferred_element_type=jnp.float32)

    y = agg + Ux
    y_ref[...] = y

    stats_ref[...] = jnp.concatenate(
        [jnp.sum(y, axis=-1, keepdims=True),
         jnp.sum(y * y, axis=-1, keepdims=True)], axis=1)


def _bn_relu_kernel(x_ref, y_ref, ss_ref, o_ref):
    scale = ss_ref[:, 0:1]
    shift = ss_ref[:, 1:2]
    o_ref[...] = jnp.maximum(x_ref[...] + y_ref[...] * scale + shift, 0.0)


def kernel(x, Uw, Ub, Vw, Vb, gamma, beta):
    B, N, C = x.shape
    xf = x.astype(jnp.float32)

    Wuv = jnp.concatenate([Uw.T, Vw.T], axis=1).astype(jnp.float32)
    buv = jnp.concatenate([Ub, Vb], axis=0).reshape(1, 2 * C).astype(jnp.float32)

    blk_x = pl.BlockSpec((None, N, C), lambda b: (b, 0, 0))

    y, stats = pl.pallas_call(
        _graph_kernel,
        out_shape=(jax.ShapeDtypeStruct((B, N, C), jnp.float32),
                   jax.ShapeDtypeStruct((B, N, 2), jnp.float32)),
        grid=(B,),
        in_specs=[blk_x,
                  pl.BlockSpec((C, 2 * C), lambda b: (0, 0)),
                  pl.BlockSpec((1, 2 * C), lambda b: (0, 0))],
        out_specs=(pl.BlockSpec((None, N, C), lambda b: (b, 0, 0)),
                   pl.BlockSpec((None, N, 2), lambda b: (b, 0, 0))),
        compiler_params=pltpu.CompilerParams(dimension_semantics=("parallel",)),
    )(xf, Wuv, buv)

    s = jnp.sum(stats, axis=0)
    inv_cnt = jnp.float32(1.0 / (B * C))
    mean = s[:, 0] * inv_cnt
    var = s[:, 1] * inv_cnt - mean * mean
    inv_std = lax.rsqrt(var + jnp.float32(_EPS))
    scale = gamma.astype(jnp.float32) * inv_std
    shift = beta.astype(jnp.float32) - mean * scale
    ss = jnp.stack([scale, shift], axis=1)

    out = pl.pallas_call(
        _bn_relu_kernel,
        out_shape=jax.ShapeDtypeStruct((B, N, C), jnp.float32),
        grid=(B,),
        in_specs=[blk_x,
                  pl.BlockSpec((None, N, C), lambda b: (b, 0, 0)),
                  pl.BlockSpec((N, 2), lambda b: (0, 0))],
        out_specs=pl.BlockSpec((None, N, C), lambda b: (b, 0, 0)),
        compiler_params=pltpu.CompilerParams(dimension_semantics=("parallel",)),
    )(xf, y, ss)
    return out


# speedup vs baseline: 1.2003x; 1.0355x over previous
import jax
import jax.numpy as jnp
from jax import lax
from jax.experimental import pallas as pl
from jax.experimental.pallas import tpu as pltpu

_K = 4
_EPS = 1e-5
_HP = lax.Precision.HIGHEST


def _graph_kernel(x_ref, wuv_ref, buv_ref, y_ref, stats_ref):
    x = x_ref[...]
    N, C = x.shape

    uv = jnp.dot(x, wuv_ref[...],
                 preferred_element_type=jnp.float32) + buv_ref[...]
    Ux = uv[:, :C]
    Vx = uv[:, C:]

    si = lax.dot_general(x, x, (((1,), (1,)), ((), ())),
                         precision=_HP,
                         preferred_element_type=jnp.float32)

    work = si
    for _ in range(_K - 1):
        m = jnp.max(work, axis=-1, keepdims=True)
        work = jnp.where(work == m, -jnp.inf, work)
    thr = jnp.max(work, axis=-1, keepdims=True)
    adj = (si >= thr).astype(jnp.float32)

    deg = jnp.sum(adj, axis=-1, keepdims=True)
    dinv = lax.rsqrt(deg)
    agg = dinv * jnp.dot(adj, dinv * Vx, preferred_element_type=jnp.float32)

    y = agg + Ux
    y_ref[...] = y.astype(jnp.bfloat16)

    part = jnp.concatenate(
        [jnp.sum(y, axis=-1, keepdims=True),
         jnp.sum(y * y, axis=-1, keepdims=True)], axis=1)

    @pl.when(pl.program_id(0) == 0)
    def _():
        stats_ref[...] = jnp.zeros_like(stats_ref)

    stats_ref[...] += part


def _bn_relu_kernel(x_ref, y_ref, s_ref, gb_ref, o_ref, *, inv_cnt):
    mean = s_ref[:, 0:1] * inv_cnt
    var = s_ref[:, 1:2] * inv_cnt - mean * mean
    inv_std = lax.rsqrt(var + jnp.float32(_EPS))
    scale = gb_ref[:, 0:1] * inv_std
    shift = gb_ref[:, 1:2] - mean * scale
    y = y_ref[...].astype(jnp.float32)
    o_ref[...] = jnp.maximum(x_ref[...] + y * scale + shift, 0.0)


def kernel(x, Uw, Ub, Vw, Vb, gamma, beta):
    import functools
    B, N, C = x.shape
    xf = x.astype(jnp.float32)

    Wuv = jnp.concatenate([Uw.T, Vw.T], axis=1).astype(jnp.float32)
    buv = jnp.concatenate([Ub, Vb], axis=0).reshape(1, 2 * C).astype(jnp.float32)
    gb = jnp.stack([gamma.astype(jnp.float32),
                    beta.astype(jnp.float32)], axis=1)

    blk_x = pl.BlockSpec((None, N, C), lambda b: (b, 0, 0))

    y, stats = pl.pallas_call(
        _graph_kernel,
        out_shape=(jax.ShapeDtypeStruct((B, N, C), jnp.bfloat16),
                   jax.ShapeDtypeStruct((N, 2), jnp.float32)),
        grid=(B,),
        in_specs=[blk_x,
                  pl.BlockSpec((C, 2 * C), lambda b: (0, 0)),
                  pl.BlockSpec((1, 2 * C), lambda b: (0, 0))],
        out_specs=(pl.BlockSpec((None, N, C), lambda b: (b, 0, 0)),
                   pl.BlockSpec((N, 2), lambda b: (0, 0))),
        compiler_params=pltpu.CompilerParams(
            dimension_semantics=("arbitrary",)),
    )(xf, Wuv, buv)

    out = pl.pallas_call(
        functools.partial(_bn_relu_kernel, inv_cnt=1.0 / (B * C)),
        out_shape=jax.ShapeDtypeStruct((B, N, C), jnp.float32),
        grid=(B,),
        in_specs=[blk_x,
                  pl.BlockSpec((None, N, C), lambda b: (b, 0, 0)),
                  pl.BlockSpec((N, 2), lambda b: (0, 0)),
                  pl.BlockSpec((N, 2), lambda b: (0, 0))],
        out_specs=pl.BlockSpec((None, N, C), lambda b: (b, 0, 0)),
        compiler_params=pltpu.CompilerParams(
            dimension_semantics=("arbitrary",)),
    )(xf, y, stats, gb)
    return out


# speedup vs baseline: 2.9664x; 2.4714x over previous
import jax
import jax.numpy as jnp
from jax import lax
from jax.experimental import pallas as pl
from jax.experimental.pallas import tpu as pltpu

_K = 4
_EPS = 1e-5


def _topk_adj(si):
    work = si
    for _ in range(_K - 1):
        m = jnp.max(work, axis=-1, keepdims=True)
        work = jnp.where(work == m, -jnp.inf, work)
    thr = jnp.max(work, axis=-1, keepdims=True)
    adj = (si >= thr).astype(jnp.float32)
    deg = jnp.sum(adj, axis=-1, keepdims=True)
    return adj, lax.rsqrt(deg)


def _hilo(x):
    hi = x.astype(jnp.bfloat16)
    lo = (x - hi.astype(jnp.float32)).astype(jnp.bfloat16)
    return hi, lo


def _gram(hi, lo, mxu):
    pltpu.matmul_push_rhs(hi, 0, mxu, transpose=True)
    pltpu.matmul_push_rhs(lo, 1, mxu, transpose=True)
    pltpu.matmul_acc_lhs(0, hi, mxu, load_staged_rhs=0)
    pltpu.matmul_acc_lhs(0, lo, mxu)
    pltpu.matmul_acc_lhs(0, hi, mxu, load_staged_rhs=1)


def _uv(wu, wv, x, mxu):
    pltpu.matmul_push_rhs(wu, 0, mxu)
    pltpu.matmul_acc_lhs(64, x, mxu, load_staged_rhs=0)
    pltpu.matmul_push_rhs(wv, 1, mxu)
    pltpu.matmul_acc_lhs(128, x, mxu, load_staged_rhs=1)


def _graph_kernel(x_ref, wu_ref, wv_ref, buv_ref, xy_ref, stats_ref):
    xs = [x_ref[i] for i in range(4)]
    N, C = xs[0].shape
    wu = wu_ref[...]
    wv = wv_ref[...]
    buv = buv_ref[...]
    bu = buv[:, :C]
    bv = buv[:, C:]

    hl = [_hilo(x) for x in xs]
    for i in range(4):
        xy_ref[i, 0] = hl[i][0]

    _gram(hl[0][0], hl[0][1], 0)
    _gram(hl[1][0], hl[1][1], 1)
    _uv(wu, wv, xs[1], 0)
    _uv(wu, wv, xs[0], 1)

    si0 = pltpu.matmul_pop(0, (N, N), jnp.float32, 0)
    si1 = pltpu.matmul_pop(0, (N, N), jnp.float32, 1)
    _gram(hl[2][0], hl[2][1], 0)
    _gram(hl[3][0], hl[3][1], 1)
    Ux1 = pltpu.matmul_pop(64, (N, C), jnp.float32, 0) + bu
    Vx1 = pltpu.matmul_pop(128, (N, C), jnp.float32, 0) + bv
    Ux0 = pltpu.matmul_pop(64, (N, C), jnp.float32, 1) + bu
    Vx0 = pltpu.matmul_pop(128, (N, C), jnp.float32, 1) + bv
    _uv(wu, wv, xs[3], 0)
    _uv(wu, wv, xs[2], 1)

    adj0, dinv0 = _topk_adj(si0)
    adj1, dinv1 = _topk_adj(si1)

    r0 = dinv0 * Vx0
    r1 = dinv1 * Vx1
    pltpu.matmul_push_rhs(r0, 0, 0)
    pltpu.matmul_acc_lhs(192, adj0, 0, load_staged_rhs=0)
    pltpu.matmul_push_rhs(r1, 0, 1)
    pltpu.matmul_acc_lhs(192, adj1, 1, load_staged_rhs=0)

    si2 = pltpu.matmul_pop(0, (N, N), jnp.float32, 0)
    si3 = pltpu.matmul_pop(0, (N, N), jnp.float32, 1)
    adj2, dinv2 = _topk_adj(si2)
    adj3, dinv3 = _topk_adj(si3)

    Ux3 = pltpu.matmul_pop(64, (N, C), jnp.float32, 0) + bu
    Vx3 = pltpu.matmul_pop(128, (N, C), jnp.float32, 0) + bv
    Ux2 = pltpu.matmul_pop(64, (N, C), jnp.float32, 1) + bu
    Vx2 = pltpu.matmul_pop(128, (N, C), jnp.float32, 1) + bv

    agg0 = dinv0 * pltpu.matmul_pop(192, (N, C), jnp.float32, 0)
    agg1 = dinv1 * pltpu.matmul_pop(192, (N, C), jnp.float32, 1)
    r2 = dinv2 * Vx2
    r3 = dinv3 * Vx3
    pltpu.matmul_push_rhs(r2, 0, 0)
    pltpu.matmul_acc_lhs(192, adj2, 0, load_staged_rhs=0)
    pltpu.matmul_push_rhs(r3, 0, 1)
    pltpu.matmul_acc_lhs(192, adj3, 1, load_staged_rhs=0)

    y0 = agg0 + Ux0
    y1 = agg1 + Ux1
    xy_ref[0, 1] = y0.astype(jnp.bfloat16)
    xy_ref[1, 1] = y1.astype(jnp.bfloat16)
    ssum = (jnp.sum(y0, axis=-1, keepdims=True) +
            jnp.sum(y1, axis=-1, keepdims=True))
    sqsum = (jnp.sum(y0 * y0, axis=-1, keepdims=True) +
             jnp.sum(y1 * y1, axis=-1, keepdims=True))

    agg2 = dinv2 * pltpu.matmul_pop(192, (N, C), jnp.float32, 0)
    agg3 = dinv3 * pltpu.matmul_pop(192, (N, C), jnp.float32, 1)
    y2 = agg2 + Ux2
    y3 = agg3 + Ux3
    xy_ref[2, 1] = y2.astype(jnp.bfloat16)
    xy_ref[3, 1] = y3.astype(jnp.bfloat16)
    ssum = (ssum + jnp.sum(y2, axis=-1, keepdims=True) +
            jnp.sum(y3, axis=-1, keepdims=True))
    sqsum = (sqsum + jnp.sum(y2 * y2, axis=-1, keepdims=True) +
             jnp.sum(y3 * y3, axis=-1, keepdims=True))

    part = jnp.concatenate([ssum, sqsum], axis=1)

    @pl.when(pl.program_id(0) == 0)
    def _():
        stats_ref[...] = jnp.zeros_like(stats_ref)

    stats_ref[...] += part


def _bn_relu_kernel(xy_ref, ss_ref, o_ref):
    scale = ss_ref[:, 0:1]
    shift = ss_ref[:, 1:2]
    x = xy_ref[:, 0].astype(jnp.float32)
    y = xy_ref[:, 1].astype(jnp.float32)
    o_ref[...] = jnp.maximum(x + y * scale + shift, 0.0)


def kernel(x, Uw, Ub, Vw, Vb, gamma, beta):
    B, N, C = x.shape
    xf = x.astype(jnp.float32)

    Wu = Uw.T.astype(jnp.float32)
    Wv = Vw.T.astype(jnp.float32)
    buv = jnp.concatenate([Ub, Vb], axis=0).reshape(1, 2 * C).astype(jnp.float32)

    xy, stats = pl.pallas_call(
        _graph_kernel,
        out_shape=(jax.ShapeDtypeStruct((B, 2, N, C), jnp.bfloat16),
                   jax.ShapeDtypeStruct((N, 2), jnp.float32)),
        grid=(B // 4,),
        in_specs=[pl.BlockSpec((4, N, C), lambda b: (b, 0, 0)),
                  pl.BlockSpec((C, C), lambda b: (0, 0)),
                  pl.BlockSpec((C, C), lambda b: (0, 0)),
                  pl.BlockSpec((1, 2 * C), lambda b: (0, 0))],
        out_specs=(pl.BlockSpec((4, 2, N, C), lambda b: (b, 0, 0, 0)),
                   pl.BlockSpec((N, 2), lambda b: (0, 0))),
        compiler_params=pltpu.CompilerParams(
            dimension_semantics=("arbitrary",)),
    )(xf, Wu, Wv, buv)

    inv_cnt = jnp.float32(1.0 / (B * C))
    mean = stats[:, 0] * inv_cnt
    var = stats[:, 1] * inv_cnt - mean * mean
    inv_std = lax.rsqrt(var + jnp.float32(_EPS))
    scale = gamma.astype(jnp.float32) * inv_std
    shift = beta.astype(jnp.float32) - mean * scale
    ss = jnp.stack([scale, shift], axis=1)

    out = pl.pallas_call(
        _bn_relu_kernel,
        out_shape=jax.ShapeDtypeStruct((B, N, C), jnp.float32),
        grid=(B // 4,),
        in_specs=[pl.BlockSpec((4, 2, N, C), lambda b: (b, 0, 0, 0)),
                  pl.BlockSpec((N, 2), lambda b: (0, 0))],
        out_specs=pl.BlockSpec((4, N, C), lambda b: (b, 0, 0)),
        compiler_params=pltpu.CompilerParams(
            dimension_semantics=("arbitrary",)),
    )(xy, ss)
    return out


# speedup vs baseline: 3.4379x; 1.1589x over previous
import jax
import jax.numpy as jnp
from jax import lax
from jax.experimental import pallas as pl
from jax.experimental.pallas import tpu as pltpu

_K = 4
_EPS = 1e-5


def _topk_adj(si):
    work = si
    for _ in range(_K - 1):
        m = jnp.max(work, axis=-1, keepdims=True)
        work = jnp.where(work == m, -jnp.inf, work)
    thr = jnp.max(work, axis=-1, keepdims=True)
    adj = (si >= thr).astype(jnp.float32)
    deg = jnp.sum(adj, axis=-1, keepdims=True)
    return adj, lax.rsqrt(deg)


def _hilo(x):
    hi = x.astype(jnp.bfloat16)
    lo = (x - hi.astype(jnp.float32)).astype(jnp.bfloat16)
    return hi, lo


def _gram(hi, lo, mxu):
    pltpu.matmul_push_rhs(hi, 0, mxu, transpose=True)
    pltpu.matmul_push_rhs(lo, 1, mxu, transpose=True)
    pltpu.matmul_acc_lhs(0, hi, mxu, load_staged_rhs=0)
    pltpu.matmul_acc_lhs(0, lo, mxu)
    pltpu.matmul_acc_lhs(0, hi, mxu, load_staged_rhs=1)


def _uv(wu, wv, x, mxu):
    pltpu.matmul_push_rhs(wu, 0, mxu)
    pltpu.matmul_acc_lhs(64, x, mxu, load_staged_rhs=0)
    pltpu.matmul_push_rhs(wv, 1, mxu)
    pltpu.matmul_acc_lhs(128, x, mxu, load_staged_rhs=1)


_NB = 8


def _graph_kernel(x_ref, wu_ref, wv_ref, buv_ref, xy_ref, stats_ref):
    P = _NB // 2
    xs = [x_ref[i] for i in range(_NB)]
    N, C = xs[0].shape
    wu = wu_ref[...]
    wv = wv_ref[...]
    buv = buv_ref[...]
    bu = buv[:, :C]
    bv = buv[:, C:]

    hl = [_hilo(x) for x in xs]
    for i in range(_NB):
        xy_ref[i, 0] = hl[i][0]

    Ux = [None] * _NB
    Vx = [None] * _NB
    adj = [None] * _NB
    dinv = [None] * _NB
    ys = [None] * _NB

    def _finish(i):
        mxu = i % 2
        agg = dinv[i] * pltpu.matmul_pop(192, (N, C), jnp.float32, mxu)
        yi = agg + Ux[i]
        xy_ref[i, 1] = yi.astype(jnp.bfloat16)
        ys[i] = yi

    def _aggregate(i):
        mxu = i % 2
        r = dinv[i] * Vx[i]
        pltpu.matmul_push_rhs(r, 0, mxu)
        pltpu.matmul_acc_lhs(192, adj[i], mxu, load_staged_rhs=0)

    sis = [None] * _NB
    for p in range(P):
        a, b = 2 * p, 2 * p + 1
        if p:
            pa, pb = a - 2, b - 2
            sis[pa] = pltpu.matmul_pop(0, (N, N), jnp.float32, 0)
            sis[pb] = pltpu.matmul_pop(0, (N, N), jnp.float32, 1)
        _gram(hl[a][0], hl[a][1], 0)
        _gram(hl[b][0], hl[b][1], 1)
        if p:
            pa, pb = a - 2, b - 2
            Ux[pb] = pltpu.matmul_pop(64, (N, C), jnp.float32, 0) + bu
            Vx[pb] = pltpu.matmul_pop(128, (N, C), jnp.float32, 0) + bv
            Ux[pa] = pltpu.matmul_pop(64, (N, C), jnp.float32, 1) + bu
            Vx[pa] = pltpu.matmul_pop(128, (N, C), jnp.float32, 1) + bv
        _uv(wu, wv, xs[b], 0)
        _uv(wu, wv, xs[a], 1)
        if p:
            pa, pb = a - 2, b - 2
            adj[pa], dinv[pa] = _topk_adj(sis[pa])
            adj[pb], dinv[pb] = _topk_adj(sis[pb])
            if p > 1:
                _finish(a - 4)
                _finish(b - 4)
            _aggregate(pa)
            _aggregate(pb)

    la, lb = _NB - 2, _NB - 1
    sis[la] = pltpu.matmul_pop(0, (N, N), jnp.float32, 0)
    sis[lb] = pltpu.matmul_pop(0, (N, N), jnp.float32, 1)
    Ux[lb] = pltpu.matmul_pop(64, (N, C), jnp.float32, 0) + bu
    Vx[lb] = pltpu.matmul_pop(128, (N, C), jnp.float32, 0) + bv
    Ux[la] = pltpu.matmul_pop(64, (N, C), jnp.float32, 1) + bu
    Vx[la] = pltpu.matmul_pop(128, (N, C), jnp.float32, 1) + bv
    adj[la], dinv[la] = _topk_adj(sis[la])
    adj[lb], dinv[lb] = _topk_adj(sis[lb])
    _finish(la - 2)
    _finish(lb - 2)
    _aggregate(la)
    _aggregate(lb)
    _finish(la)
    _finish(lb)

    ssum = sum((jnp.sum(yi, axis=-1, keepdims=True) for yi in ys),
               jnp.zeros((N, 1), jnp.float32))
    sqsum = sum((jnp.sum(yi * yi, axis=-1, keepdims=True) for yi in ys),
                jnp.zeros((N, 1), jnp.float32))
    part = jnp.concatenate([ssum, sqsum], axis=1)

    @pl.when(pl.program_id(0) == 0)
    def _():
        stats_ref[...] = jnp.zeros_like(stats_ref)

    stats_ref[...] += part


def _bn_relu_kernel(xy_ref, ss_ref, o_ref):
    scale = ss_ref[:, 0:1]
    shift = ss_ref[:, 1:2]
    x = xy_ref[:, 0].astype(jnp.float32)
    y = xy_ref[:, 1].astype(jnp.float32)
    o_ref[...] = jnp.maximum(x + y * scale + shift, 0.0)


def kernel(x, Uw, Ub, Vw, Vb, gamma, beta):
    B, N, C = x.shape
    xf = x.astype(jnp.float32)

    Wu = Uw.T.astype(jnp.float32)
    Wv = Vw.T.astype(jnp.float32)
    buv = jnp.concatenate([Ub, Vb], axis=0).reshape(1, 2 * C).astype(jnp.float32)

    xy, stats = pl.pallas_call(
        _graph_kernel,
        out_shape=(jax.ShapeDtypeStruct((B, 2, N, C), jnp.bfloat16),
                   jax.ShapeDtypeStruct((N, 2), jnp.float32)),
        grid=(B // _NB,),
        in_specs=[pl.BlockSpec((_NB, N, C), lambda b: (b, 0, 0)),
                  pl.BlockSpec((C, C), lambda b: (0, 0)),
                  pl.BlockSpec((C, C), lambda b: (0, 0)),
                  pl.BlockSpec((1, 2 * C), lambda b: (0, 0))],
        out_specs=(pl.BlockSpec((_NB, 2, N, C), lambda b: (b, 0, 0, 0)),
                   pl.BlockSpec((N, 2), lambda b: (0, 0))),
        compiler_params=pltpu.CompilerParams(
            dimension_semantics=("arbitrary",)),
    )(xf, Wu, Wv, buv)

    inv_cnt = jnp.float32(1.0 / (B * C))
    mean = stats[:, 0] * inv_cnt
    var = stats[:, 1] * inv_cnt - mean * mean
    inv_std = lax.rsqrt(var + jnp.float32(_EPS))
    scale = gamma.astype(jnp.float32) * inv_std
    shift = beta.astype(jnp.float32) - mean * scale
    ss = jnp.stack([scale, shift], axis=1)

    out = pl.pallas_call(
        _bn_relu_kernel,
        out_shape=jax.ShapeDtypeStruct((B, N, C), jnp.float32),
        grid=(B // _NB,),
        in_specs=[pl.BlockSpec((_NB, 2, N, C), lambda b: (b, 0, 0, 0)),
                  pl.BlockSpec((N, 2), lambda b: (0, 0))],
        out_specs=pl.BlockSpec((_NB, N, C), lambda b: (b, 0, 0)),
        compiler_params=pltpu.CompilerParams(
            dimension_semantics=("arbitrary",)),
    )(xy, ss)
    return out


# speedup vs baseline: 3.5571x; 1.0347x over previous
import jax
import jax.numpy as jnp
from jax import lax
from jax.experimental import pallas as pl
from jax.experimental.pallas import tpu as pltpu

_K = 4
_EPS = 1e-5


def _topk_adj(si):
    work = si
    for _ in range(_K - 1):
        m = jnp.max(work, axis=-1, keepdims=True)
        work = jnp.where(work == m, -jnp.inf, work)
    thr = jnp.max(work, axis=-1, keepdims=True)
    adj = (si >= thr).astype(jnp.float32)
    deg = jnp.sum(adj, axis=-1, keepdims=True)
    return adj, lax.rsqrt(deg)


def _hilo(x):
    hi = x.astype(jnp.bfloat16)
    lo = (x - hi.astype(jnp.float32)).astype(jnp.bfloat16)
    return hi, lo


def _gram(hi, lo, mxu):
    pltpu.matmul_push_rhs(hi, 0, mxu, transpose=True)
    pltpu.matmul_push_rhs(lo, 1, mxu, transpose=True)
    pltpu.matmul_acc_lhs(0, hi, mxu, load_staged_rhs=0)
    pltpu.matmul_acc_lhs(0, lo, mxu)
    pltpu.matmul_acc_lhs(0, hi, mxu, load_staged_rhs=1)


def _uv(wu, wv, x, mxu):
    pltpu.matmul_push_rhs(wu, 0, mxu)
    pltpu.matmul_acc_lhs(64, x, mxu, load_staged_rhs=0)
    pltpu.matmul_push_rhs(wv, 1, mxu)
    pltpu.matmul_acc_lhs(128, x, mxu, load_staged_rhs=1)


_NB = 16


def _graph_kernel(x_ref, wu_ref, wv_ref, buv_ref, xy_ref, stats_ref):
    P = _NB // 2
    xs = [x_ref[i] for i in range(_NB)]
    N, C = xs[0].shape
    wu = wu_ref[...]
    wv = wv_ref[...]
    buv = buv_ref[...]
    bu = buv[:, :C]
    bv = buv[:, C:]

    hl = [_hilo(x) for x in xs]
    for i in range(_NB):
        xy_ref[i, 0] = hl[i][0]

    Ux = [None] * _NB
    Vx = [None] * _NB
    adj = [None] * _NB
    dinv = [None] * _NB
    stat = [None] * _NB

    def _finish(i):
        mxu = i % 2
        agg = dinv[i] * pltpu.matmul_pop(192, (N, C), jnp.float32, mxu)
        yi = agg + Ux[i]
        xy_ref[i, 1] = yi.astype(jnp.bfloat16)
        stat[i] = (jnp.sum(yi, axis=-1, keepdims=True),
                   jnp.sum(yi * yi, axis=-1, keepdims=True))

    def _aggregate(i):
        mxu = i % 2
        r = dinv[i] * Vx[i]
        pltpu.matmul_push_rhs(r, 0, mxu)
        pltpu.matmul_acc_lhs(192, adj[i], mxu, load_staged_rhs=0)

    sis = [None] * _NB
    for p in range(P):
        a, b = 2 * p, 2 * p + 1
        if p:
            pa, pb = a - 2, b - 2
            sis[pa] = pltpu.matmul_pop(0, (N, N), jnp.float32, 0)
            sis[pb] = pltpu.matmul_pop(0, (N, N), jnp.float32, 1)
        _gram(hl[a][0], hl[a][1], 0)
        _gram(hl[b][0], hl[b][1], 1)
        if p:
            pa, pb = a - 2, b - 2
            Ux[pb] = pltpu.matmul_pop(64, (N, C), jnp.float32, 0) + bu
            Vx[pb] = pltpu.matmul_pop(128, (N, C), jnp.float32, 0) + bv
            Ux[pa] = pltpu.matmul_pop(64, (N, C), jnp.float32, 1) + bu
            Vx[pa] = pltpu.matmul_pop(128, (N, C), jnp.float32, 1) + bv
        _uv(wu, wv, xs[b], 0)
        _uv(wu, wv, xs[a], 1)
        if p:
            pa, pb = a - 2, b - 2
            adj[pa], dinv[pa] = _topk_adj(sis[pa])
            adj[pb], dinv[pb] = _topk_adj(sis[pb])
            if p > 1:
                _finish(a - 4)
                _finish(b - 4)
            _aggregate(pa)
            _aggregate(pb)

    la, lb = _NB - 2, _NB - 1
    sis[la] = pltpu.matmul_pop(0, (N, N), jnp.float32, 0)
    sis[lb] = pltpu.matmul_pop(0, (N, N), jnp.float32, 1)
    Ux[lb] = pltpu.matmul_pop(64, (N, C), jnp.float32, 0) + bu
    Vx[lb] = pltpu.matmul_pop(128, (N, C), jnp.float32, 0) + bv
    Ux[la] = pltpu.matmul_pop(64, (N, C), jnp.float32, 1) + bu
    Vx[la] = pltpu.matmul_pop(128, (N, C), jnp.float32, 1) + bv
    adj[la], dinv[la] = _topk_adj(sis[la])
    adj[lb], dinv[lb] = _topk_adj(sis[lb])
    _finish(la - 2)
    _finish(lb - 2)
    _aggregate(la)
    _aggregate(lb)
    _finish(la)
    _finish(lb)

    ssum = sum((s for s, _ in stat), jnp.zeros((N, 1), jnp.float32))
    sqsum = sum((q for _, q in stat), jnp.zeros((N, 1), jnp.float32))
    part = jnp.concatenate([ssum, sqsum], axis=1)

    @pl.when(pl.program_id(0) == 0)
    def _():
        stats_ref[...] = jnp.zeros_like(stats_ref)

    stats_ref[...] += part


def _bn_relu_kernel(xy_ref, ss_ref, o_ref):
    scale = ss_ref[:, 0:1]
    shift = ss_ref[:, 1:2]
    x = xy_ref[:, 0].astype(jnp.float32)
    y = xy_ref[:, 1].astype(jnp.float32)
    o_ref[...] = jnp.maximum(x + y * scale + shift, 0.0)


def kernel(x, Uw, Ub, Vw, Vb, gamma, beta):
    B, N, C = x.shape
    xf = x.astype(jnp.float32)

    Wu = Uw.T.astype(jnp.float32)
    Wv = Vw.T.astype(jnp.float32)
    buv = jnp.concatenate([Ub, Vb], axis=0).reshape(1, 2 * C).astype(jnp.float32)

    xy, stats = pl.pallas_call(
        _graph_kernel,
        out_shape=(jax.ShapeDtypeStruct((B, 2, N, C), jnp.bfloat16),
                   jax.ShapeDtypeStruct((N, 2), jnp.float32)),
        grid=(B // _NB,),
        in_specs=[pl.BlockSpec((_NB, N, C), lambda b: (b, 0, 0)),
                  pl.BlockSpec((C, C), lambda b: (0, 0)),
                  pl.BlockSpec((C, C), lambda b: (0, 0)),
                  pl.BlockSpec((1, 2 * C), lambda b: (0, 0))],
        out_specs=(pl.BlockSpec((_NB, 2, N, C), lambda b: (b, 0, 0, 0)),
                   pl.BlockSpec((N, 2), lambda b: (0, 0))),
        compiler_params=pltpu.CompilerParams(
            dimension_semantics=("arbitrary",)),
    )(xf, Wu, Wv, buv)

    inv_cnt = jnp.float32(1.0 / (B * C))
    mean = stats[:, 0] * inv_cnt
    var = stats[:, 1] * inv_cnt - mean * mean
    inv_std = lax.rsqrt(var + jnp.float32(_EPS))
    scale = gamma.astype(jnp.float32) * inv_std
    shift = beta.astype(jnp.float32) - mean * scale
    ss = jnp.stack([scale, shift], axis=1)

    out = pl.pallas_call(
        _bn_relu_kernel,
        out_shape=jax.ShapeDtypeStruct((B, N, C), jnp.float32),
        grid=(B // _NB,),
        in_specs=[pl.BlockSpec((_NB, 2, N, C), lambda b: (b, 0, 0, 0)),
                  pl.BlockSpec((N, 2), lambda b: (0, 0))],
        out_specs=pl.BlockSpec((_NB, N, C), lambda b: (b, 0, 0)),
        compiler_params=pltpu.CompilerParams(
            dimension_semantics=("arbitrary",)),
    )(xy, ss)
    return out
